# baseline (speedup 1.0000x reference)
import functools

import jax
import jax.numpy as jnp
from jax import lax
from jax.experimental import pallas as pl
from jax.experimental.pallas import tpu as pltpu
from jax.experimental.pallas import tpu_sc as plsc

B = 64
N = 576
D = 768

NC = 2
NS = 16
NW = NC * NS
BPW = B // NW

CH = 16
NCH = N // CH
NPAIR = NCH // 2
LANES = 16
LB = 8
NLB = D // (LANES * LB)

_mesh = plsc.VectorSubcoreMesh(core_axis_name="c", subcore_axis_name="s")


@functools.partial(
    pl.kernel,
    mesh=_mesh,
    out_type=jax.ShapeDtypeStruct((B, N + 1, D), jnp.float32),
    scratch_types=[
        pltpu.VMEM((CH, D), jnp.float32),
        pltpu.VMEM((CH, D), jnp.float32),
        pltpu.VMEM((CH, D), jnp.float32),
        pltpu.VMEM((CH, D), jnp.float32),
        pltpu.VMEM((CH, D), jnp.float32),
        pltpu.VMEM((CH, D), jnp.float32),
        pltpu.VMEM((CH, D), jnp.float32),
        pltpu.VMEM((CH, D), jnp.float32),
        pltpu.VMEM((BPW, D), jnp.float32),
        pltpu.VMEM((1, D), jnp.float32),
        pltpu.VMEM_SHARED((N + 1, D), jnp.float32),
        pltpu.SemaphoreType.DMA,
        pltpu.SemaphoreType.DMA,
        pltpu.SemaphoreType.DMA,
        pltpu.SemaphoreType.DMA,
        pltpu.SemaphoreType.DMA,
        pltpu.SemaphoreType.DMA,
        pltpu.SemaphoreType.DMA,
        pltpu.SemaphoreType.DMA,
    ],
)
def _encode(patch_hbm, pos_hbm, out_hbm,
            pa0, pa1, pb0, pb1, oa, ob, q0, q1, cbuf, tbuf, spos,
            s_pa0, s_pa1, s_pb0, s_pb1, s_oa, s_ob, s_q0, s_q1):
    sid = lax.axis_index("s")
    wid = sid * NC + lax.axis_index("c")
    b0 = wid * BPW

    @pl.when(sid < 14)
    def _():
        off = pl.multiple_of(sid * 40, 8)
        pltpu.sync_copy(pos_hbm.at[pl.ds(off, 40)], spos.at[pl.ds(off, 40)])

    @pl.when(sid == 14)
    def _():
        pltpu.sync_copy(pos_hbm.at[pl.ds(560, 17)], spos.at[pl.ds(560, 17)])

    plsc.subcore_barrier()

    zero = jnp.zeros((LANES,), jnp.float32)
    for bb in range(BPW):
        for j in range(D // LANES):
            cbuf[bb, pl.ds(j * LANES, LANES)] = zero

    def start_in(c, pat_a, pat_b, qb, s_a, s_b, s_q):
        r0 = c * CH
        pltpu.make_async_copy(spos.at[pl.ds(r0, CH)], qb, s_q).start()
        pltpu.make_async_copy(patch_hbm.at[b0, pl.ds(r0, CH)], pat_a, s_a).start()
        pltpu.make_async_copy(patch_hbm.at[b0 + 1, pl.ds(r0, CH)], pat_b, s_b).start()

    def compute(pbuf, obuf, qb, bb):
        def jj_body(jj, _):
            base = jj * (LANES * LB)
            for u in range(LB):
                sl = pl.ds(base + u * LANES, LANES)
                obuf[0, sl] = qb[0, sl] + cbuf[bb, sl]
                for r in range(1, CH):
                    obuf[r, sl] = qb[r, sl] + pbuf[r - 1, sl]
                cbuf[bb, sl] = pbuf[CH - 1, sl]
            return 0

        lax.fori_loop(0, NLB, jj_body, 0)

    def drain_out(obuf, bq, sem):
        pltpu.make_async_copy(obuf, out_hbm.at[b0 + bq, pl.ds(0, CH)], sem).wait()

    def do_chunk(i2, c, pat_a, pat_b, qb, s_a, s_b, s_q, odd):
        r0 = c * CH
        pltpu.make_async_copy(spos.at[pl.ds(r0, CH)], qb, s_q).wait()
        pltpu.make_async_copy(patch_hbm.at[b0, pl.ds(r0, CH)], pat_a, s_a).wait()

        if odd:
            drain_out(oa, 0, s_oa)
        else:
            @pl.when(i2 > 0)
            def _():
                drain_out(oa, 0, s_oa)

        compute(pat_a, oa, qb, 0)
        pltpu.make_async_copy(oa, out_hbm.at[b0, pl.ds(r0, CH)], s_oa).start()

        pltpu.make_async_copy(patch_hbm.at[b0 + 1, pl.ds(r0, CH)], pat_b, s_b).wait()

        if odd:
            drain_out(ob, 1, s_ob)
        else:
            @pl.when(i2 > 0)
            def _():
                drain_out(ob, 1, s_ob)

        compute(pat_b, ob, qb, 1)
        pltpu.make_async_copy(ob, out_hbm.at[b0 + 1, pl.ds(r0, CH)], s_ob).start()

    start_in(0, pa0, pb0, q0, s_pa0, s_pb0, s_q0)

    def pair_body(i2, _):
        e = 2 * i2
        start_in(e + 1, pa1, pb1, q1, s_pa1, s_pb1, s_q1)
        do_chunk(i2, e, pa0, pb0, q0, s_pa0, s_pb0, s_q0, odd=False)

        @pl.when(i2 < NPAIR - 1)
        def _():
            start_in(e + 2, pa0, pb0, q0, s_pa0, s_pb0, s_q0)

        do_chunk(i2, e + 1, pa1, pb1, q1, s_pa1, s_pb1, s_q1, odd=True)
        return 0

    lax.fori_loop(0, NPAIR, pair_body, 0)

    drain_out(oa, 0, s_oa)
    drain_out(ob, 1, s_ob)

    pltpu.sync_copy(spos.at[pl.ds(N, 1)], tbuf)
    for bb, dst in ((0, pa0), (1, pb0)):
        for j in range(D // LANES):
            sl = pl.ds(j * LANES, LANES)
            dst[0, sl] = tbuf[0, sl] + cbuf[bb, sl]
        pltpu.sync_copy(dst.at[pl.ds(0, 1)], out_hbm.at[b0 + bb, pl.ds(N, 1)])


def kernel(patch, pos_emb):
    return _encode(patch, pos_emb)
